# baseline (speedup 1.0000x reference)
import functools

import jax
import jax.numpy as jnp
import numpy as np
from jax.experimental import pallas as pl
from jax.experimental.pallas import tpu as pltpu
from jax.experimental.pallas import tpu_sc as plsc

C = 128
OUT_C = 64
H = 250
W = 250
KS = 5
ST = 5
HS = H // ST
WS = W // ST
NP = HS * WS
HEADS = 8
K = 7
S = K + 1
NPAD = 2560
BN = 512
NBLK = NPAD // BN
PD = ST * ST * C
WPAD = 256
FH = H + 3

_NEG = -1e30


def _static_graph():
    ii, jj = np.meshgrid(np.arange(0, H - H % ST, ST, dtype=np.float32),
                         np.arange(0, W - W % ST, ST, dtype=np.float32),
                         indexing='ij')
    cen = np.stack([ii.reshape(-1), jj.reshape(-1)], axis=1)
    dist = np.sqrt(((cen[:, None, :] - cen[None, :, :]) ** 2).sum(-1))
    dist = dist.astype(np.float32)
    order = np.argsort(dist, axis=1, kind='stable')[:, :K].astype(np.int64)
    vals = np.take_along_axis(dist, order, axis=1)
    rows = np.arange(NP, dtype=np.int64)[:, None]
    ok = (order >= rows) & (vals != 0)
    nbr = np.zeros((NPAD, S), np.int32)
    msk = np.zeros((NPAD, S), np.float32)
    nbr[:NP, :K] = np.where(ok, order, rows)
    msk[:NP, :K] = ok.astype(np.float32)
    nbr[:NP, K] = rows[:, 0]
    msk[:NP, K] = 1.0
    return nbr, msk


_NBR2, _MSK2 = _static_graph()
_NBR2T = np.ascontiguousarray(_NBR2.T).reshape(-1)


def _sel_mats():
    sel = np.zeros((W, W), np.float32)
    selt = np.zeros((W, WPAD), np.float32)
    for b in range(ST):
        for j in range(WS):
            sel[5 * j + b, b * WS + j] = 1.0
            selt[b * WS + j, 1 + 5 * j + b] = 1.0
    return sel, selt


_SEL, _SELT = _sel_mats()


def _leaky(v):
    return jnp.where(v >= 0, v, 0.2 * v)


def _elu(v):
    return jnp.where(v > 0, v, jnp.exp(jnp.minimum(v, 0.0)) - 1.0)


def _dot(a, b):
    return jax.lax.dot_general(a, b, (((1,), (0,)), ((), ())),
                               preferred_element_type=jnp.float32)


_PR = 5
_PB = _PR * WS
_RB = _PR * ST
_EG = HS // _PR


def _emb_body(x_ref, w1c_ref, b1_ref, f2_ref, b2_ref, sel_ref, out_ref):
    xabs = [_dot(x_ref[:, 0, rr, :], sel_ref[...]) for rr in range(_RB)]
    zt = jnp.zeros((C, _PB), jnp.float32)
    for a in range(ST):
        for b in range(ST):
            xg = jnp.concatenate(
                [xabs[5 * g + a][:, b * WS:(b + 1) * WS]
                 for g in range(_PR)], axis=1)
            k = a * ST + b
            zt = zt + _dot(w1c_ref[k * C:(k + 1) * C, :], xg)
    zt = jnp.maximum(zt + b1_ref[...], 0.0)
    et = jnp.maximum(_dot(f2_ref[...], zt) + b2_ref[...], 0.0)
    out_ref[0] = et.T


def _emb_call(x4, w1c, b1c, fc2, b2c, sel):
    return pl.pallas_call(
        _emb_body,
        grid=(_EG,),
        in_specs=[
            pl.BlockSpec((C, 1, _RB, W), lambda i: (0, i, 0, 0)),
            pl.BlockSpec((ST * ST * C, C), lambda i: (0, 0)),
            pl.BlockSpec((C, 1), lambda i: (0, 0)),
            pl.BlockSpec((C, C), lambda i: (0, 0)),
            pl.BlockSpec((C, 1), lambda i: (0, 0)),
            pl.BlockSpec((W, W), lambda i: (0, 0)),
        ],
        out_specs=pl.BlockSpec((1, _PB, C), lambda i: (i, 0, 0)),
        out_shape=jax.ShapeDtypeStruct((_EG, _PB, C), jnp.float32),
    )(x4, w1c, b1c, fc2, b2c, sel)


def _topk_body(emb_ref, embt_ref, nbr_ref, msk_ref):
    pid = pl.program_id(0)
    xb = emb_ref[pl.ds(pid * BN, BN), :]
    adj = _dot(xb, embt_ref[...])
    col = jax.lax.broadcasted_iota(jnp.int32, (BN, NPAD), 1)
    rowid = pid * BN + jax.lax.broadcasted_iota(jnp.int32, (BN, 1), 0)
    work = jnp.where(col < NP, adj, _NEG)
    idxs = []
    msks = []
    for _ in range(K):
        m = jnp.max(work, axis=1, keepdims=True)
        sel = jnp.where((work == m) & (m > _NEG), col, NPAD)
        idx = jnp.min(sel, axis=1, keepdims=True)
        valid = (rowid <= idx) & (m != 0) & (idx < NPAD)
        idxs.append(jnp.where(valid, idx, rowid))
        msks.append(jnp.where(valid, 1.0, 0.0))
        work = jnp.where(col == idx, _NEG, work)
    idxs.append(rowid)
    msks.append(jnp.ones((BN, 1), jnp.float32))
    nbr_ref[...] = jnp.concatenate(idxs, axis=1)
    msk_ref[...] = jnp.concatenate(msks, axis=1)


def _topk_call(emb, embt):
    return pl.pallas_call(
        _topk_body,
        grid=(NBLK,),
        in_specs=[
            pl.BlockSpec((NPAD, C), lambda i: (0, 0)),
            pl.BlockSpec((C, NPAD), lambda i: (0, 0)),
        ],
        out_specs=[
            pl.BlockSpec((BN, S), lambda i: (i, 0)),
            pl.BlockSpec((BN, S), lambda i: (i, 0)),
        ],
        out_shape=[
            jax.ShapeDtypeStruct((NPAD, S), jnp.int32),
            jax.ShapeDtypeStruct((NPAD, S), jnp.float32),
        ],
    )(emb, embt)


def _sc_gather(table, idx):
    info = plsc.get_sparse_core_info()
    nc, ns = info.num_cores, info.num_subcores
    nw = nc * ns
    B = idx.shape[0]
    bpw = B // nw
    mesh = plsc.VectorSubcoreMesh(core_axis_name="c", subcore_axis_name="s")

    @functools.partial(
        pl.kernel, mesh=mesh,
        out_type=jax.ShapeDtypeStruct((B, C), jnp.float32),
        scratch_types=[
            pltpu.VMEM((bpw,), jnp.int32),
            pltpu.VMEM((bpw, C), jnp.float32),
            pltpu.SemaphoreType.DMA,
        ],
    )
    def k(table_hbm, idx_hbm, out_hbm, idx_v, rows_v, sem):
        wid = jax.lax.axis_index("s") * nc + jax.lax.axis_index("c")
        base = wid * bpw
        pltpu.sync_copy(idx_hbm.at[pl.ds(base, bpw)], idx_v)
        pltpu.async_copy(table_hbm.at[idx_v], rows_v, sem).wait()
        pltpu.sync_copy(rows_v, out_hbm.at[pl.ds(base, bpw)])

    return k(table, idx)


def _att_hi_math(hb, gh_ref, msk_ref, wl_ref, bl_ref, wr_ref, br_ref,
                 atta_ref, bias_ref):
    xr = _dot(hb, wr_ref[...]) + br_ref[...]
    gs = []
    logits = []
    for j in range(S):
        g = gh_ref[j]
        xl = _dot(g, wl_ref[...]) + bl_ref[...]
        e = _leaky(xl + xr)
        lg = _dot(e, atta_ref[...])
        mj = msk_ref[:, j:j + 1]
        gs.append(g)
        logits.append(jnp.where(mj > 0, lg, _NEG))
    m = logits[0]
    for j in range(1, S):
        m = jnp.maximum(m, logits[j])
    exs = [jnp.exp(lg - m) for lg in logits]
    den = exs[0]
    for j in range(1, S):
        den = den + exs[j]
    rden = 1.0 / (den + 1e-16)
    alphas = [ex * rden for ex in exs]
    asum = alphas[0]
    for j in range(1, S):
        asum = asum + alphas[j]
    outs = []
    for hh in range(HEADS):
        gh = alphas[0][:, hh:hh + 1] * gs[0]
        for j in range(1, S):
            gh = gh + alphas[j][:, hh:hh + 1] * gs[j]
        oh = _dot(gh, wl_ref[:, hh * C:(hh + 1) * C])
        oh = oh + asum[:, hh:hh + 1] * bl_ref[:, hh * C:(hh + 1) * C]
        outs.append(oh)
    out = jnp.concatenate(outs, axis=1)
    return _elu(out + bias_ref[...])


def _att_hi2_body(h_ref, gh1_ref, msk1_ref, wl1_ref, bl1_ref, wr1_ref,
                  br1_ref, atta1_ref, bias1_ref, gh3_ref, msk3_ref,
                  wl3_ref, bl3_ref, wr3_ref, br3_ref, atta3_ref,
                  bias3_ref, out1_ref, out3_ref):
    hb = h_ref[...]
    out1_ref[...] = _att_hi_math(hb, gh1_ref, msk1_ref, wl1_ref, bl1_ref,
                                 wr1_ref, br1_ref, atta1_ref, bias1_ref)
    out3_ref[...] = _att_hi_math(hb, gh3_ref, msk3_ref, wl3_ref, bl3_ref,
                                 wr3_ref, br3_ref, atta3_ref, bias3_ref)


def _att_hi2_call(h, gh1, msk1, w1, gh3, msk3, w3):
    D = HEADS * C
    wspec = [
        pl.BlockSpec((C, D), lambda i: (0, 0)),
        pl.BlockSpec((1, D), lambda i: (0, 0)),
        pl.BlockSpec((C, D), lambda i: (0, 0)),
        pl.BlockSpec((1, D), lambda i: (0, 0)),
        pl.BlockSpec((D, HEADS), lambda i: (0, 0)),
        pl.BlockSpec((1, D), lambda i: (0, 0)),
    ]
    gspec = [
        pl.BlockSpec((S, BN, C), lambda i: (0, i, 0)),
        pl.BlockSpec((BN, S), lambda i: (i, 0)),
    ]
    return pl.pallas_call(
        _att_hi2_body,
        grid=(NBLK,),
        in_specs=[pl.BlockSpec((BN, C), lambda i: (i, 0))]
        + gspec + wspec + gspec + wspec,
        out_specs=[
            pl.BlockSpec((BN, D), lambda i: (i, 0)),
            pl.BlockSpec((BN, D), lambda i: (i, 0)),
        ],
        out_shape=[
            jax.ShapeDtypeStruct((NPAD, D), jnp.float32),
            jax.ShapeDtypeStruct((NPAD, D), jnp.float32),
        ],
    )(h, gh1, msk1, *w1, gh3, msk3, *w3)


def _xlr2_body(h1_ref, wl2_ref, bl2_ref, wr2_ref, br2_ref,
               h2_ref, wl4_ref, bl4_ref, wr4_ref, br4_ref,
               xl2_ref, xr2_ref, xl4_ref, xr4_ref):
    h1b = h1_ref[...]
    xl2_ref[...] = _dot(h1b, wl2_ref[...]) + bl2_ref[...]
    xr2_ref[...] = _dot(h1b, wr2_ref[...]) + br2_ref[...]
    h2b = h2_ref[...]
    xl4_ref[...] = _dot(h2b, wl4_ref[...]) + bl4_ref[...]
    xr4_ref[...] = _dot(h2b, wr4_ref[...]) + br4_ref[...]


def _xlr2_call(h1, w2, h2, w4):
    D = h1.shape[1]
    spec = [
        pl.BlockSpec((BN, D), lambda i: (i, 0)),
        pl.BlockSpec((D, C), lambda i: (0, 0)),
        pl.BlockSpec((1, C), lambda i: (0, 0)),
        pl.BlockSpec((D, C), lambda i: (0, 0)),
        pl.BlockSpec((1, C), lambda i: (0, 0)),
    ]
    ospec = pl.BlockSpec((BN, C), lambda i: (i, 0))
    return pl.pallas_call(
        _xlr2_body,
        grid=(NBLK,),
        in_specs=spec + spec,
        out_specs=[ospec] * 4,
        out_shape=[jax.ShapeDtypeStruct((NPAD, C), jnp.float32)] * 4,
    )(h1, *w2, h2, *w4)


def _att_lo_math(gl_ref, xr_ref, msk_ref, att_ref, bias_ref):
    xr = xr_ref[...]
    gs = []
    logits = []
    for j in range(S):
        g = gl_ref[j]
        e = _leaky(g + xr)
        lg = jnp.sum(e * att_ref[...], axis=1, keepdims=True)
        mj = msk_ref[:, j:j + 1]
        gs.append(g)
        logits.append(jnp.where(mj > 0, lg, _NEG))
    m = logits[0]
    for j in range(1, S):
        m = jnp.maximum(m, logits[j])
    exs = [jnp.exp(lg - m) for lg in logits]
    den = exs[0]
    for j in range(1, S):
        den = den + exs[j]
    rden = 1.0 / (den + 1e-16)
    out = (exs[0] * rden) * gs[0]
    for j in range(1, S):
        out = out + (exs[j] * rden) * gs[j]
    return _elu(out + bias_ref[...])


def _att_lo2_body(gl2_ref, xr2_ref, msk1_ref, att2_ref, bias2_ref,
                  gl4_ref, xr4_ref, msk3_ref, att4_ref, bias4_ref,
                  out1_ref, out2_ref):
    out1_ref[...] = _att_lo_math(gl2_ref, xr2_ref, msk1_ref, att2_ref,
                                 bias2_ref)
    out2_ref[...] = _att_lo_math(gl4_ref, xr4_ref, msk3_ref, att4_ref,
                                 bias4_ref)


def _att_lo2_call(gl2, xr2, msk1, att2, bias2, gl4, xr4, msk3, att4,
                  bias4):
    spec = [
        pl.BlockSpec((S, BN, C), lambda i: (0, i, 0)),
        pl.BlockSpec((BN, C), lambda i: (i, 0)),
        pl.BlockSpec((BN, S), lambda i: (i, 0)),
        pl.BlockSpec((1, C), lambda i: (0, 0)),
        pl.BlockSpec((1, C), lambda i: (0, 0)),
    ]
    ospec = pl.BlockSpec((BN, C), lambda i: (i, 0))
    return pl.pallas_call(
        _att_lo2_body,
        grid=(NBLK,),
        in_specs=spec + spec,
        out_specs=[ospec] * 2,
        out_shape=[jax.ShapeDtypeStruct((NPAD, C), jnp.float32)] * 2,
    )(gl2, xr2, msk1, att2, bias2, gl4, xr4, msk3, att4, bias4)


def _up_body(h_ref, h2_ref, fa_ref, fb_ref, fbias_ref, x_ref, ctwt_ref,
             ctb_ref, wint_ref, inb_ref, sel_ref, selt_ref, out_ref):
    p = pl.program_id(0)
    valid = (p >= 1) & (p <= _EG)

    @pl.when(valid)
    def _():
        h2 = h2_ref[0]
        hc = _dot(h_ref[0], fa_ref[...]) + _dot(h2, fb_ref[...]) \
            + fbias_ref[...]
        hout = jnp.maximum(hc, 0.0) + h2
        hbt = hout.T
        up = _dot(ctwt_ref[...], hbt)
        for rr in range(_RB):
            g, a = rr // ST, rr % ST
            xab = _dot(x_ref[:, 0, rr, :], sel_ref[...])
            upt = jnp.concatenate(
                [up[(a * ST + b) * C:(a * ST + b + 1) * C,
                    g * WS:(g + 1) * WS] for b in range(ST)], axis=1)
            feat = jnp.maximum(xab + upt + ctb_ref[...], 0.0)
            it = _dot(wint_ref[...], feat) + inb_ref[...]
            out_ref[:, 0, rr, :] = _dot(it, selt_ref[...])

    @pl.when(jnp.logical_not(valid))
    def _():
        out_ref[...] = jnp.zeros((OUT_C, 1, _RB, WPAD), jnp.float32)


def _up_call(h3, h23, fa, fb, fbias, x4, ctwt, ctb, wint, inb, sel, selt):
    def _pm(p):
        return jnp.minimum(jnp.maximum(p - 1, 0), _EG - 1)
    return pl.pallas_call(
        _up_body,
        grid=(_EG + 2,),
        in_specs=[
            pl.BlockSpec((1, _PB, C), lambda p: (_pm(p), 0, 0)),
            pl.BlockSpec((1, _PB, C), lambda p: (_pm(p), 0, 0)),
            pl.BlockSpec((C, C), lambda p: (0, 0)),
            pl.BlockSpec((C, C), lambda p: (0, 0)),
            pl.BlockSpec((1, C), lambda p: (0, 0)),
            pl.BlockSpec((C, 1, _RB, W), lambda p: (0, _pm(p), 0, 0)),
            pl.BlockSpec((ST * ST * C, C), lambda p: (0, 0)),
            pl.BlockSpec((C, 1), lambda p: (0, 0)),
            pl.BlockSpec((OUT_C, C), lambda p: (0, 0)),
            pl.BlockSpec((OUT_C, 1), lambda p: (0, 0)),
            pl.BlockSpec((W, W), lambda p: (0, 0)),
            pl.BlockSpec((W, WPAD), lambda p: (0, 0)),
        ],
        out_specs=pl.BlockSpec((OUT_C, 1, _RB, WPAD),
                               lambda p: (0, p, 0, 0)),
        out_shape=jax.ShapeDtypeStruct((OUT_C, _EG + 2, _RB, WPAD),
                                       jnp.float32),
    )(h3, h23, fa, fb, fbias, x4, ctwt, ctb, wint, inb, sel, selt)


_CONV_BL = 6400
_CONV_NB = H * WPAD // _CONV_BL
_CONV_WIN = _CONV_BL + 128


def _conv3_body(flat_ref, w_ref, lyb_ref, out_ref):
    pid = pl.program_id(0)
    wins = [flat_ref[:, pl.ds((25 * pid + _RB - 1 + di) * WPAD, _CONV_WIN)]
            for di in range(3)]
    acc = None
    for dj in range(3):
        p = _dot(w_ref[pl.ds(dj * OUT_C, OUT_C), :], wins[0])
        for di in (1, 2):
            p = p + _dot(w_ref[pl.ds((di * 3 + dj) * OUT_C, OUT_C), :],
                         wins[di])
        sl = pltpu.roll(p, _CONV_WIN - dj, 1)[:, :_CONV_BL] if dj \
            else p[:, :_CONV_BL]
        acc = sl if acc is None else acc + sl
    out_ref[...] = acc + lyb_ref[...]


def _conv3_call(flat, wflat, lyb):
    return pl.pallas_call(
        _conv3_body,
        grid=(_CONV_NB,),
        in_specs=[
            pl.BlockSpec((OUT_C, (_EG + 2) * _RB * WPAD), lambda i: (0, 0)),
            pl.BlockSpec((9 * OUT_C, OUT_C), lambda i: (0, 0)),
            pl.BlockSpec((OUT_C, 1), lambda i: (0, 0)),
        ],
        out_specs=pl.BlockSpec((OUT_C, _CONV_BL), lambda i: (0, i)),
        out_shape=jax.ShapeDtypeStruct((OUT_C, H * WPAD), jnp.float32),
    )(flat, wflat, lyb)


def kernel(x, conv1_w, conv1_b, fc2_w, fc2_b, fc3_w, fc3_b,
           g1_wl, g1_bl, g1_wr, g1_br, g1_att, g1_bias,
           g2_wl, g2_bl, g2_wr, g2_br, g2_att, g2_bias,
           g3_wl, g3_bl, g3_wr, g3_br, g3_att, g3_bias,
           g4_wl, g4_bl, g4_wr, g4_br, g4_att, g4_bias,
           ct_w, ct_b, in_w, in_b, ly_w, ly_b):
    f32 = jnp.float32
    x4 = x.reshape(C, _EG, _RB, W)
    sel = jnp.asarray(_SEL)
    selt = jnp.asarray(_SELT)

    w1c = conv1_w.transpose(2, 3, 0, 1).reshape(PD, C)
    emb3 = _emb_call(x4, w1c, conv1_b[:, None], fc2_w, fc2_b[:, None], sel)
    emb = jnp.concatenate([emb3.reshape(NP, C),
                           jnp.zeros((NPAD - NP, C), f32)], axis=0)

    nbr1, msk1 = _topk_call(emb, emb.T)
    nbr1t = nbr1.T.reshape(-1)
    nbr2t = jnp.asarray(_NBR2T)
    msk2 = jnp.asarray(_MSK2)

    lane = jnp.arange(HEADS * C)
    atta1 = jnp.zeros((HEADS * C, HEADS), f32).at[lane, lane // C].set(
        g1_att.reshape(-1))
    atta3 = jnp.zeros((HEADS * C, HEADS), f32).at[lane, lane // C].set(
        g3_att.reshape(-1))

    gh1 = _sc_gather(emb, nbr1t).reshape(S, NPAD, C)
    gh3 = _sc_gather(emb, nbr2t).reshape(S, NPAD, C)

    w1set = (g1_wl, g1_bl[None], g1_wr, g1_br[None], atta1, g1_bias[None])
    w3set = (g3_wl, g3_bl[None], g3_wr, g3_br[None], atta3, g3_bias[None])
    h1, h2a = _att_hi2_call(emb, gh1, msk1, w1set, gh3, msk2, w3set)

    xl2, xr2, xl4, xr4 = _xlr2_call(
        h1, (g2_wl, g2_bl[None], g2_wr, g2_br[None]),
        h2a, (g4_wl, g4_bl[None], g4_wr, g4_br[None]))
    gl2 = _sc_gather(xl2, nbr1t).reshape(S, NPAD, C)
    gl4 = _sc_gather(xl4, nbr2t).reshape(S, NPAD, C)
    h, h2 = _att_lo2_call(gl2, xr2, msk1, g2_att, g2_bias[None],
                          gl4, xr4, msk2, g4_att, g4_bias[None])

    h3 = h[:NP].reshape(_EG, _PB, C)
    h23 = h2[:NP].reshape(_EG, _PB, C)
    ctwt = ct_w.transpose(2, 3, 1, 0).reshape(PD, C)
    wint = in_w.reshape(OUT_C, C)
    buf = _up_call(h3, h23, fc3_w[:, :C].T, fc3_w[:, C:].T, fc3_b[None],
                   x4, ctwt, ct_b[:, None], wint, in_b[:, None], sel, selt)
    flat = buf.reshape(OUT_C, (_EG + 2) * _RB * WPAD)

    wflat = ly_w.transpose(2, 3, 0, 1).reshape(9 * OUT_C, OUT_C)
    acc = _conv3_call(flat, wflat, ly_b[:, None])
    out = acc.reshape(OUT_C, H, WPAD)[:, :, :W]
    return out[None]


# speedup vs baseline: 1.1262x; 1.1262x over previous
import functools

import jax
import jax.numpy as jnp
import numpy as np
from jax.experimental import pallas as pl
from jax.experimental.pallas import tpu as pltpu
from jax.experimental.pallas import tpu_sc as plsc

C = 128
OUT_C = 64
H = 250
W = 250
KS = 5
ST = 5
HS = H // ST
WS = W // ST
NP = HS * WS
HEADS = 8
K = 7
S = K + 1
NPAD = 2560
BN = 512
NBLK = NPAD // BN
PD = ST * ST * C
WPAD = 256
FH = H + 3

_NEG = -1e30


def _static_graph():
    ii, jj = np.meshgrid(np.arange(0, H - H % ST, ST, dtype=np.float32),
                         np.arange(0, W - W % ST, ST, dtype=np.float32),
                         indexing='ij')
    cen = np.stack([ii.reshape(-1), jj.reshape(-1)], axis=1)
    dist = np.sqrt(((cen[:, None, :] - cen[None, :, :]) ** 2).sum(-1))
    dist = dist.astype(np.float32)
    order = np.argsort(dist, axis=1, kind='stable')[:, :K].astype(np.int64)
    vals = np.take_along_axis(dist, order, axis=1)
    rows = np.arange(NP, dtype=np.int64)[:, None]
    ok = (order >= rows) & (vals != 0)
    nbr = np.zeros((NPAD, S), np.int32)
    msk = np.zeros((NPAD, S), np.float32)
    nbr[:NP, :K] = np.where(ok, order, rows)
    msk[:NP, :K] = ok.astype(np.float32)
    nbr[:NP, K] = rows[:, 0]
    msk[:NP, K] = 1.0
    return nbr, msk


_NBR2, _MSK2 = _static_graph()
_NBR2T = np.ascontiguousarray(_NBR2.T).reshape(-1)


def _sel_mats():
    sel = np.zeros((W, W), np.float32)
    selt = np.zeros((W, WPAD), np.float32)
    for b in range(ST):
        for j in range(WS):
            sel[5 * j + b, b * WS + j] = 1.0
            selt[b * WS + j, 1 + 5 * j + b] = 1.0
    return sel, selt


_SEL, _SELT = _sel_mats()


def _leaky(v):
    return jnp.where(v >= 0, v, 0.2 * v)


def _elu(v):
    return jnp.where(v > 0, v, jnp.exp(jnp.minimum(v, 0.0)) - 1.0)


def _dot(a, b):
    return jax.lax.dot_general(a, b, (((1,), (0,)), ((), ())),
                               preferred_element_type=jnp.float32)


_PR = 5
_PB = _PR * WS
_RB = _PR * ST
_EG = HS // _PR


def _emb_body(x_ref, w1c_ref, b1_ref, f2_ref, b2_ref, sel_ref, out_ref):
    xabs = [_dot(x_ref[:, 0, rr, :], sel_ref[...]) for rr in range(_RB)]
    zt = jnp.zeros((C, _PB), jnp.float32)
    for a in range(ST):
        for b in range(ST):
            xg = jnp.concatenate(
                [xabs[5 * g + a][:, b * WS:(b + 1) * WS]
                 for g in range(_PR)], axis=1)
            k = a * ST + b
            zt = zt + _dot(w1c_ref[k * C:(k + 1) * C, :], xg)
    zt = jnp.maximum(zt + b1_ref[...], 0.0)
    et = jnp.maximum(_dot(f2_ref[...], zt) + b2_ref[...], 0.0)
    out_ref[0] = et.T


def _emb_call(x4, w1c, b1c, fc2, b2c, sel):
    return pl.pallas_call(
        _emb_body,
        grid=(_EG,),
        in_specs=[
            pl.BlockSpec((C, 1, _RB, W), lambda i: (0, i, 0, 0)),
            pl.BlockSpec((ST * ST * C, C), lambda i: (0, 0)),
            pl.BlockSpec((C, 1), lambda i: (0, 0)),
            pl.BlockSpec((C, C), lambda i: (0, 0)),
            pl.BlockSpec((C, 1), lambda i: (0, 0)),
            pl.BlockSpec((W, W), lambda i: (0, 0)),
        ],
        out_specs=pl.BlockSpec((1, _PB, C), lambda i: (i, 0, 0)),
        out_shape=jax.ShapeDtypeStruct((_EG, _PB, C), jnp.float32),
    )(x4, w1c, b1c, fc2, b2c, sel)


def _topk_body(emb_ref, embt_ref, nbr_ref, msk_ref):
    pid = pl.program_id(0)
    xb = emb_ref[pl.ds(pid * BN, BN), :]
    adj = _dot(xb, embt_ref[...])
    col = jax.lax.broadcasted_iota(jnp.int32, (BN, NPAD), 1)
    rowid = pid * BN + jax.lax.broadcasted_iota(jnp.int32, (BN, 1), 0)
    work = jnp.where(col < NP, adj, _NEG)
    idxs = []
    msks = []
    for _ in range(K):
        m = jnp.max(work, axis=1, keepdims=True)
        sel = jnp.where((work == m) & (m > _NEG), col, NPAD)
        idx = jnp.min(sel, axis=1, keepdims=True)
        valid = (rowid <= idx) & (m != 0) & (idx < NPAD)
        idxs.append(jnp.where(valid, idx, rowid))
        msks.append(jnp.where(valid, 1.0, 0.0))
        work = jnp.where(col == idx, _NEG, work)
    idxs.append(rowid)
    msks.append(jnp.ones((BN, 1), jnp.float32))
    nbr_ref[...] = jnp.concatenate(idxs, axis=1)
    msk_ref[...] = jnp.concatenate(msks, axis=1)


def _topk_call(emb, embt):
    return pl.pallas_call(
        _topk_body,
        grid=(NBLK,),
        in_specs=[
            pl.BlockSpec((NPAD, C), lambda i: (0, 0)),
            pl.BlockSpec((C, NPAD), lambda i: (0, 0)),
        ],
        out_specs=[
            pl.BlockSpec((BN, S), lambda i: (i, 0)),
            pl.BlockSpec((BN, S), lambda i: (i, 0)),
        ],
        out_shape=[
            jax.ShapeDtypeStruct((NPAD, S), jnp.int32),
            jax.ShapeDtypeStruct((NPAD, S), jnp.float32),
        ],
    )(emb, embt)


def _sc_gather(table, idx):
    info = plsc.get_sparse_core_info()
    nc, ns = info.num_cores, info.num_subcores
    nw = nc * ns
    B = idx.shape[0]
    bpw = B // nw
    mesh = plsc.VectorSubcoreMesh(core_axis_name="c", subcore_axis_name="s")

    @functools.partial(
        pl.kernel, mesh=mesh,
        out_type=jax.ShapeDtypeStruct((B, C), jnp.float32),
        scratch_types=[
            pltpu.VMEM((bpw,), jnp.int32),
            pltpu.VMEM((bpw, C), jnp.float32),
            pltpu.SemaphoreType.DMA,
        ],
    )
    def k(table_hbm, idx_hbm, out_hbm, idx_v, rows_v, sem):
        wid = jax.lax.axis_index("s") * nc + jax.lax.axis_index("c")
        base = wid * bpw
        pltpu.sync_copy(idx_hbm.at[pl.ds(base, bpw)], idx_v)
        pltpu.async_copy(table_hbm.at[idx_v], rows_v, sem).wait()
        pltpu.sync_copy(rows_v, out_hbm.at[pl.ds(base, bpw)])

    return k(table, idx)


def _att_hi_math(hb, gh_ref, msk_ref, wl_ref, bl_ref, wr_ref, br_ref,
                 atta_ref, bias_ref):
    xr = _dot(hb, wr_ref[...]) + br_ref[...]
    gs = []
    logits = []
    for j in range(S):
        g = gh_ref[j]
        xl = _dot(g, wl_ref[...]) + bl_ref[...]
        e = _leaky(xl + xr)
        lg = _dot(e, atta_ref[...])
        mj = msk_ref[:, j:j + 1]
        gs.append(g)
        logits.append(jnp.where(mj > 0, lg, _NEG))
    m = logits[0]
    for j in range(1, S):
        m = jnp.maximum(m, logits[j])
    exs = [jnp.exp(lg - m) for lg in logits]
    den = exs[0]
    for j in range(1, S):
        den = den + exs[j]
    rden = 1.0 / (den + 1e-16)
    alphas = [ex * rden for ex in exs]
    asum = alphas[0]
    for j in range(1, S):
        asum = asum + alphas[j]
    outs = []
    for hh in range(HEADS):
        gh = alphas[0][:, hh:hh + 1] * gs[0]
        for j in range(1, S):
            gh = gh + alphas[j][:, hh:hh + 1] * gs[j]
        oh = _dot(gh, wl_ref[:, hh * C:(hh + 1) * C])
        oh = oh + asum[:, hh:hh + 1] * bl_ref[:, hh * C:(hh + 1) * C]
        outs.append(oh)
    out = jnp.concatenate(outs, axis=1)
    return _elu(out + bias_ref[...])


def _att_hi_body(h_ref, gh_ref, msk_ref, wl_ref, bl_ref, wr_ref,
                 br_ref, atta_ref, bias_ref, out_ref):
    out_ref[...] = _att_hi_math(h_ref[...], gh_ref, msk_ref, wl_ref,
                                bl_ref, wr_ref, br_ref, atta_ref,
                                bias_ref)


def _att_hi_call(h, gh, msk, w):
    D = HEADS * C
    return pl.pallas_call(
        _att_hi_body,
        grid=(NBLK,),
        in_specs=[
            pl.BlockSpec((BN, C), lambda i: (i, 0)),
            pl.BlockSpec((S, BN, C), lambda i: (0, i, 0)),
            pl.BlockSpec((BN, S), lambda i: (i, 0)),
            pl.BlockSpec((C, D), lambda i: (0, 0)),
            pl.BlockSpec((1, D), lambda i: (0, 0)),
            pl.BlockSpec((C, D), lambda i: (0, 0)),
            pl.BlockSpec((1, D), lambda i: (0, 0)),
            pl.BlockSpec((D, HEADS), lambda i: (0, 0)),
            pl.BlockSpec((1, D), lambda i: (0, 0)),
        ],
        out_specs=pl.BlockSpec((BN, D), lambda i: (i, 0)),
        out_shape=jax.ShapeDtypeStruct((NPAD, D), jnp.float32),
    )(h, gh, msk, *w)


def _xlr_body(h_ref, wl_ref, bl_ref, wr_ref, br_ref, xl_ref, xr_ref):
    hb = h_ref[...]
    xl_ref[...] = _dot(hb, wl_ref[...]) + bl_ref[...]
    xr_ref[...] = _dot(hb, wr_ref[...]) + br_ref[...]


def _xlr_call(h, w):
    D = h.shape[1]
    ospec = pl.BlockSpec((BN, C), lambda i: (i, 0))
    return pl.pallas_call(
        _xlr_body,
        grid=(NBLK,),
        in_specs=[
            pl.BlockSpec((BN, D), lambda i: (i, 0)),
            pl.BlockSpec((D, C), lambda i: (0, 0)),
            pl.BlockSpec((1, C), lambda i: (0, 0)),
            pl.BlockSpec((D, C), lambda i: (0, 0)),
            pl.BlockSpec((1, C), lambda i: (0, 0)),
        ],
        out_specs=[ospec] * 2,
        out_shape=[jax.ShapeDtypeStruct((NPAD, C), jnp.float32)] * 2,
    )(h, *w)


def _att_lo_math(gl_ref, xr_ref, msk_ref, att_ref, bias_ref):
    xr = xr_ref[...]
    gs = []
    logits = []
    for j in range(S):
        g = gl_ref[j]
        e = _leaky(g + xr)
        lg = jnp.sum(e * att_ref[...], axis=1, keepdims=True)
        mj = msk_ref[:, j:j + 1]
        gs.append(g)
        logits.append(jnp.where(mj > 0, lg, _NEG))
    m = logits[0]
    for j in range(1, S):
        m = jnp.maximum(m, logits[j])
    exs = [jnp.exp(lg - m) for lg in logits]
    den = exs[0]
    for j in range(1, S):
        den = den + exs[j]
    rden = 1.0 / (den + 1e-16)
    out = (exs[0] * rden) * gs[0]
    for j in range(1, S):
        out = out + (exs[j] * rden) * gs[j]
    return _elu(out + bias_ref[...])


def _att_lo_body(gl_ref, xr_ref, msk_ref, att_ref, bias_ref, out_ref):
    out_ref[...] = _att_lo_math(gl_ref, xr_ref, msk_ref, att_ref,
                                bias_ref)


def _att_lo_call(gl, xr, msk, att, bias):
    return pl.pallas_call(
        _att_lo_body,
        grid=(NBLK,),
        in_specs=[
            pl.BlockSpec((S, BN, C), lambda i: (0, i, 0)),
            pl.BlockSpec((BN, C), lambda i: (i, 0)),
            pl.BlockSpec((BN, S), lambda i: (i, 0)),
            pl.BlockSpec((1, C), lambda i: (0, 0)),
            pl.BlockSpec((1, C), lambda i: (0, 0)),
        ],
        out_specs=pl.BlockSpec((BN, C), lambda i: (i, 0)),
        out_shape=jax.ShapeDtypeStruct((NPAD, C), jnp.float32),
    )(gl, xr, msk, att, bias)


def _up_body(h_ref, h2_ref, fa_ref, fb_ref, fbias_ref, x_ref, ctwt_ref,
             ctb_ref, wint_ref, inb_ref, sel_ref, selt_ref, out_ref):
    p = pl.program_id(0)
    valid = (p >= 1) & (p <= _EG)

    @pl.when(valid)
    def _():
        h2 = h2_ref[0]
        hc = _dot(h_ref[0], fa_ref[...]) + _dot(h2, fb_ref[...]) \
            + fbias_ref[...]
        hout = jnp.maximum(hc, 0.0) + h2
        hbt = hout.T
        up = _dot(ctwt_ref[...], hbt)
        for rr in range(_RB):
            g, a = rr // ST, rr % ST
            xab = _dot(x_ref[:, 0, rr, :], sel_ref[...])
            upt = jnp.concatenate(
                [up[(a * ST + b) * C:(a * ST + b + 1) * C,
                    g * WS:(g + 1) * WS] for b in range(ST)], axis=1)
            feat = jnp.maximum(xab + upt + ctb_ref[...], 0.0)
            it = _dot(wint_ref[...], feat) + inb_ref[...]
            out_ref[:, 0, rr, :] = _dot(it, selt_ref[...])

    @pl.when(jnp.logical_not(valid))
    def _():
        out_ref[...] = jnp.zeros((OUT_C, 1, _RB, WPAD), jnp.float32)


def _up_call(h3, h23, fa, fb, fbias, x4, ctwt, ctb, wint, inb, sel, selt):
    def _pm(p):
        return jnp.minimum(jnp.maximum(p - 1, 0), _EG - 1)
    return pl.pallas_call(
        _up_body,
        grid=(_EG + 2,),
        in_specs=[
            pl.BlockSpec((1, _PB, C), lambda p: (_pm(p), 0, 0)),
            pl.BlockSpec((1, _PB, C), lambda p: (_pm(p), 0, 0)),
            pl.BlockSpec((C, C), lambda p: (0, 0)),
            pl.BlockSpec((C, C), lambda p: (0, 0)),
            pl.BlockSpec((1, C), lambda p: (0, 0)),
            pl.BlockSpec((C, 1, _RB, W), lambda p: (0, _pm(p), 0, 0)),
            pl.BlockSpec((ST * ST * C, C), lambda p: (0, 0)),
            pl.BlockSpec((C, 1), lambda p: (0, 0)),
            pl.BlockSpec((OUT_C, C), lambda p: (0, 0)),
            pl.BlockSpec((OUT_C, 1), lambda p: (0, 0)),
            pl.BlockSpec((W, W), lambda p: (0, 0)),
            pl.BlockSpec((W, WPAD), lambda p: (0, 0)),
        ],
        out_specs=pl.BlockSpec((OUT_C, 1, _RB, WPAD),
                               lambda p: (0, p, 0, 0)),
        out_shape=jax.ShapeDtypeStruct((OUT_C, _EG + 2, _RB, WPAD),
                                       jnp.float32),
    )(h3, h23, fa, fb, fbias, x4, ctwt, ctb, wint, inb, sel, selt)


_CONV_BL = 6400
_CONV_NB = H * WPAD // _CONV_BL
_CONV_WIN = _CONV_BL + 128


def _conv3_body(flat_ref, w_ref, lyb_ref, out_ref):
    pid = pl.program_id(0)
    wins = [flat_ref[:, pl.ds((25 * pid + _RB - 1 + di) * WPAD, _CONV_WIN)]
            for di in range(3)]
    acc = None
    for dj in range(3):
        p = _dot(w_ref[pl.ds(dj * OUT_C, OUT_C), :], wins[0])
        for di in (1, 2):
            p = p + _dot(w_ref[pl.ds((di * 3 + dj) * OUT_C, OUT_C), :],
                         wins[di])
        sl = pltpu.roll(p, _CONV_WIN - dj, 1)[:, :_CONV_BL] if dj \
            else p[:, :_CONV_BL]
        acc = sl if acc is None else acc + sl
    out_ref[...] = acc + lyb_ref[...]


def _conv3_call(flat, wflat, lyb):
    return pl.pallas_call(
        _conv3_body,
        grid=(_CONV_NB,),
        in_specs=[
            pl.BlockSpec((OUT_C, (_EG + 2) * _RB * WPAD), lambda i: (0, 0)),
            pl.BlockSpec((9 * OUT_C, OUT_C), lambda i: (0, 0)),
            pl.BlockSpec((OUT_C, 1), lambda i: (0, 0)),
        ],
        out_specs=pl.BlockSpec((OUT_C, _CONV_BL), lambda i: (0, i)),
        out_shape=jax.ShapeDtypeStruct((OUT_C, H * WPAD), jnp.float32),
    )(flat, wflat, lyb)


def kernel(x, conv1_w, conv1_b, fc2_w, fc2_b, fc3_w, fc3_b,
           g1_wl, g1_bl, g1_wr, g1_br, g1_att, g1_bias,
           g2_wl, g2_bl, g2_wr, g2_br, g2_att, g2_bias,
           g3_wl, g3_bl, g3_wr, g3_br, g3_att, g3_bias,
           g4_wl, g4_bl, g4_wr, g4_br, g4_att, g4_bias,
           ct_w, ct_b, in_w, in_b, ly_w, ly_b):
    f32 = jnp.float32
    x4 = x.reshape(C, _EG, _RB, W)
    sel = jnp.asarray(_SEL)
    selt = jnp.asarray(_SELT)

    w1c = conv1_w.transpose(2, 3, 0, 1).reshape(PD, C)
    emb3 = _emb_call(x4, w1c, conv1_b[:, None], fc2_w, fc2_b[:, None], sel)
    emb = jnp.concatenate([emb3.reshape(NP, C),
                           jnp.zeros((NPAD - NP, C), f32)], axis=0)

    nbr1, msk1 = _topk_call(emb, emb.T)
    nbr1t = nbr1.T.reshape(-1)
    nbr2t = jnp.asarray(_NBR2T)
    msk2 = jnp.asarray(_MSK2)

    lane = jnp.arange(HEADS * C)
    atta1 = jnp.zeros((HEADS * C, HEADS), f32).at[lane, lane // C].set(
        g1_att.reshape(-1))
    atta3 = jnp.zeros((HEADS * C, HEADS), f32).at[lane, lane // C].set(
        g3_att.reshape(-1))

    gh1 = _sc_gather(emb, nbr1t).reshape(S, NPAD, C)
    gh3 = _sc_gather(emb, nbr2t).reshape(S, NPAD, C)

    w1set = (g1_wl, g1_bl[None], g1_wr, g1_br[None], atta1, g1_bias[None])
    w3set = (g3_wl, g3_bl[None], g3_wr, g3_br[None], atta3, g3_bias[None])
    h1 = _att_hi_call(emb, gh1, msk1, w1set)
    xl2, xr2 = _xlr_call(h1, (g2_wl, g2_bl[None], g2_wr, g2_br[None]))
    gl2 = _sc_gather(xl2, nbr1t).reshape(S, NPAD, C)
    h = _att_lo_call(gl2, xr2, msk1, g2_att, g2_bias[None])

    h2a = _att_hi_call(emb, gh3, msk2, w3set)
    xl4, xr4 = _xlr_call(h2a, (g4_wl, g4_bl[None], g4_wr, g4_br[None]))
    gl4 = _sc_gather(xl4, nbr2t).reshape(S, NPAD, C)
    h2 = _att_lo_call(gl4, xr4, msk2, g4_att, g4_bias[None])

    h3 = h[:NP].reshape(_EG, _PB, C)
    h23 = h2[:NP].reshape(_EG, _PB, C)
    ctwt = ct_w.transpose(2, 3, 1, 0).reshape(PD, C)
    wint = in_w.reshape(OUT_C, C)
    buf = _up_call(h3, h23, fc3_w[:, :C].T, fc3_w[:, C:].T, fc3_b[None],
                   x4, ctwt, ct_b[:, None], wint, in_b[:, None], sel, selt)
    flat = buf.reshape(OUT_C, (_EG + 2) * _RB * WPAD)

    wflat = ly_w.transpose(2, 3, 0, 1).reshape(9 * OUT_C, OUT_C)
    acc = _conv3_call(flat, wflat, ly_b[:, None])
    out = acc.reshape(OUT_C, H, WPAD)[:, :, :W]
    return out[None]
